# baseline (speedup 1.0000x reference)
import functools

import jax
import jax.numpy as jnp
from jax import lax
from jax.experimental import pallas as pl
from jax.experimental.pallas import tpu as pltpu
from jax.experimental.pallas import tpu_sc as plsc

EXPERT_DIM = 512
NUM_EXPERTS = 246
BATCH = 16384

NUM_CORES = 2
NUM_SUBCORES = 16
NUM_WORKERS = NUM_CORES * NUM_SUBCORES

SC_ROWS = 8192
TC_ROWS = BATCH - SC_ROWS
B_PER_W = SC_ROWS // NUM_WORKERS
CHUNK = 64
NCHUNK = B_PER_W // CHUNK
TC_BLOCK = 512
TC_GRID = TC_ROWS // TC_BLOCK


def _transform_body(table_ref, w_ref, b_ref, out_ref):
    out_ref[...] = lax.dot_general(
        table_ref[...], w_ref[...], (((1,), (1,)), ((), ())),
        preferred_element_type=jnp.float32,
        precision=lax.Precision.HIGHEST,
    ) + b_ref[...]


def _transform(table, W, b):
    return pl.pallas_call(
        _transform_body,
        out_shape=jax.ShapeDtypeStruct((NUM_EXPERTS, EXPERT_DIM), jnp.float32),
    )(table, W, b.reshape(1, EXPERT_DIM))


_MESH = plsc.VectorSubcoreMesh(core_axis_name="c", subcore_axis_name="s")


@functools.partial(
    pl.kernel,
    mesh=_MESH,
    out_type=jax.ShapeDtypeStruct((SC_ROWS, EXPERT_DIM), jnp.float32),
    scratch_types=[
        pltpu.VMEM((B_PER_W,), jnp.int32),
        pltpu.VMEM((CHUNK, EXPERT_DIM), jnp.float32),
        pltpu.VMEM((CHUNK, EXPERT_DIM), jnp.float32),
        pltpu.SemaphoreType.DMA,
        pltpu.SemaphoreType.DMA,
        pltpu.SemaphoreType.DMA,
        pltpu.SemaphoreType.DMA,
    ],
)
def _gather(tab_hbm, idx_hbm, out_hbm, idx_v, rows0, rows1, g0, g1, s0, s1):
    wid = lax.axis_index("s") * NUM_CORES + lax.axis_index("c")
    base = wid * B_PER_W
    pltpu.sync_copy(idx_hbm.at[pl.ds(base, B_PER_W)], idx_v)
    bufs, gsem, ssem = (rows0, rows1), (g0, g1), (s0, s1)

    def start_gather(c, buf, sem):
        return pltpu.async_copy(
            tab_hbm.at[idx_v.at[pl.ds(c * CHUNK, CHUNK)]], buf, sem
        )

    def start_store(c, buf, sem):
        return pltpu.async_copy(
            buf, out_hbm.at[pl.ds(base + c * CHUNK, CHUNK)], sem
        )

    sh = [None, None]
    for c in range(NCHUNK):
        cur = c & 1
        if sh[cur] is not None:
            sh[cur].wait()
            sh[cur] = None
        start_gather(c, bufs[cur], gsem[cur]).wait()
        sh[cur] = start_store(c, bufs[cur], ssem[cur])
    for h in sh:
        if h is not None:
            h.wait()


def _onehot_body(idx_ref, t_ref, out_ref):
    idx = idx_ref[0, 0, :]
    iota = lax.broadcasted_iota(jnp.int32, (TC_BLOCK, NUM_EXPERTS), 1)
    oh = (idx[:, None] == iota).astype(jnp.float32)
    out_ref[...] = lax.dot_general(
        oh, t_ref[...], (((1,), (0,)), ((), ())),
        preferred_element_type=jnp.float32,
    )


def _onehot_rows(idx_tc, t):
    idx3 = idx_tc.reshape(TC_GRID, 1, TC_BLOCK)
    return pl.pallas_call(
        _onehot_body,
        grid=(TC_GRID,),
        in_specs=[
            pl.BlockSpec((1, 1, TC_BLOCK), lambda r: (r, 0, 0)),
            pl.BlockSpec((NUM_EXPERTS, EXPERT_DIM), lambda r: (0, 0)),
        ],
        out_specs=pl.BlockSpec((TC_BLOCK, EXPERT_DIM), lambda r: (r, 0)),
        out_shape=jax.ShapeDtypeStruct((TC_ROWS, EXPERT_DIM), jnp.float32),
    )(idx3, t)


def kernel(expert_id, table, W, b):
    t = _transform(table, W, b)
    idx = expert_id.astype(jnp.int32)
    sc_part = _gather(t, idx[:SC_ROWS])
    tc_part = _onehot_rows(idx[SC_ROWS:], t)
    return jnp.concatenate([sc_part, tc_part], axis=0)


# speedup vs baseline: 1.1937x; 1.1937x over previous
import functools

import jax
import jax.numpy as jnp
from jax import lax
from jax.experimental import pallas as pl
from jax.experimental.pallas import tpu as pltpu
from jax.experimental.pallas import tpu_sc as plsc

EXPERT_DIM = 512
NUM_EXPERTS = 246
BATCH = 16384

NUM_CORES = 2
NUM_SUBCORES = 16
NUM_WORKERS = NUM_CORES * NUM_SUBCORES
B_PER_W = BATCH // NUM_WORKERS
CHUNK = 64
NCHUNK = B_PER_W // CHUNK


def _transform_body(table_ref, w_ref, b_ref, out_ref):
    out_ref[...] = lax.dot_general(
        table_ref[...], w_ref[...], (((1,), (1,)), ((), ())),
        preferred_element_type=jnp.float32,
        precision=lax.Precision.HIGHEST,
    ) + b_ref[...]


def _transform(table, W, b):
    return pl.pallas_call(
        _transform_body,
        out_shape=jax.ShapeDtypeStruct((NUM_EXPERTS, EXPERT_DIM), jnp.float32),
    )(table, W, b.reshape(1, EXPERT_DIM))


_MESH = plsc.VectorSubcoreMesh(core_axis_name="c", subcore_axis_name="s")


@functools.partial(
    pl.kernel,
    mesh=_MESH,
    out_type=jax.ShapeDtypeStruct((BATCH, EXPERT_DIM), jnp.float32),
    scratch_types=[
        pltpu.VMEM((B_PER_W,), jnp.int32),
        pltpu.VMEM((CHUNK, EXPERT_DIM), jnp.float32),
        pltpu.VMEM((CHUNK, EXPERT_DIM), jnp.float32),
        pltpu.SemaphoreType.DMA,
        pltpu.SemaphoreType.DMA,
        pltpu.SemaphoreType.DMA,
        pltpu.SemaphoreType.DMA,
    ],
)
def _gather(tab_hbm, idx_hbm, out_hbm, idx_v, rows0, rows1, g0, g1, s0, s1):
    wid = lax.axis_index("s") * NUM_CORES + lax.axis_index("c")
    base = wid * B_PER_W
    pltpu.sync_copy(idx_hbm.at[pl.ds(base, B_PER_W)], idx_v)
    bufs, gsem, ssem = (rows0, rows1), (g0, g1), (s0, s1)

    def start_gather(c, buf, sem):
        return pltpu.async_copy(
            tab_hbm.at[idx_v.at[pl.ds(c * CHUNK, CHUNK)]], buf, sem
        )

    def start_store(c, buf, sem):
        return pltpu.async_copy(
            buf, out_hbm.at[pl.ds(base + c * CHUNK, CHUNK)], sem
        )

    sh = [None, None]
    for c in range(NCHUNK):
        cur = c & 1
        if sh[cur] is not None:
            sh[cur].wait()
            sh[cur] = None
        start_gather(c, bufs[cur], gsem[cur]).wait()
        sh[cur] = start_store(c, bufs[cur], ssem[cur])
    for h in sh:
        if h is not None:
            h.wait()


def kernel(expert_id, table, W, b):
    t = _transform(table, W, b)
    return _gather(t, expert_id.astype(jnp.int32))
